# Initial kernel scaffold; baseline (speedup 1.0000x reference)
import jax
import jax.numpy as jnp
from jax.experimental import pallas as pl


def kernel(mem, idx, val, read_idx):
    raise NotImplementedError("write your pallas kernel here")


# speedup vs baseline: 3.0215x; 3.0215x over previous
import functools

import jax
import jax.numpy as jnp
from jax import lax
from jax.experimental import pallas as pl
from jax.experimental.pallas import tpu as pltpu
from jax.experimental.pallas import tpu_sc as plsc

NC = 2
NS = 16
L = 16
NW = NC * NS

KPASS = 3
CC = 128


def _sc_body(M, B, D, mem_hbm, idx_hbm, val_hbm, ridx_hbm, out_hbm,
             widx_v, cslot_v, cval_v, bufA, bufB, bufC,
             marker_sh, sem_rows, sem_val):
    cid = lax.axis_index("c")
    sid = lax.axis_index("s")
    wid = sid * NC + cid
    RB = B // NW
    WB = B // NS
    rbase = wid * RB
    wbase = sid * WB

    jvals_v = bufA.at[pl.ds(0, WB)]
    w_v = bufA.at[pl.ds(WB, WB)]
    ridx_v = bufA.at[pl.ds(2 * WB, RB + L)]
    rw_v = bufB.at[pl.ds(0, RB)]
    gidx_v = bufB.at[pl.ds(RB, RB)]
    t_v = bufB.at[pl.ds(2 * RB, RB)]
    gbuf_v = bufB.at[pl.ds(3 * RB, RB + L)]
    dbuf_v = bufB.at[pl.ds(4 * RB + L, RB + L)]
    imp_s = bufC.at[pl.ds(0, WB + L)]
    imp_j = bufC.at[pl.ds(WB + L, WB + L)]

    pltpu.sync_copy(ridx_hbm.at[pl.ds(rbase, RB)], ridx_v.at[pl.ds(0, RB)])

    def issue_row(j, carry):
        r = ridx_v[pl.ds(j, L)][0]
        pltpu.async_copy(mem_hbm.at[r], out_hbm.at[rbase + j], sem_rows)
        return carry

    lax.fori_loop(0, RB, issue_row, jnp.int32(0))

    pltpu.sync_copy(idx_hbm.at[pl.ds(wbase, WB)], widx_v)
    for v in range(WB // L):
        jvals_v[pl.ds(v * L, L)] = lax.iota(jnp.int32, L) + (wbase + v * L)

    pltpu.sync_copy(jvals_v, marker_sh.at[widx_v])
    plsc.subcore_barrier()

    lanes = lax.iota(jnp.int32, L)
    for _ in range(KPASS):
        pltpu.sync_copy(marker_sh.at[widx_v], w_v)
        plsc.subcore_barrier()
        nc = jnp.int32(0)
        for v in range(WB // L):
            sl = pl.ds(v * L, L)
            jv = jvals_v[sl]
            m = jv > w_v[sl]
            hinc = plsc.cumsum(m.astype(jnp.int32))
            pos = nc + hinc - 1
            plsc.store_scatter(imp_s, [pos], widx_v[sl], mask=m)
            plsc.store_scatter(imp_j, [pos], jv, mask=m)
            nc = nc + jnp.sum(m.astype(jnp.int32), axis=0)

        def round_body(r, carry):
            for k in range(CC // L):
                base = r * CC + k * L
                ids = lanes + base
                sv = imp_s[pl.ds(base, L)]
                jv2 = imp_j[pl.ds(base, L)]
                cslot_v[pl.ds(k * L, L)] = jnp.where(
                    ids < nc, sv, jnp.int32(M) + lanes + (k * L))
                cval_v[pl.ds(k * L, L)] = jv2
            pltpu.sync_copy(cval_v, marker_sh.at[cslot_v])
            return carry

        lax.fori_loop(0, (nc + (CC - 1)) // CC, round_body, jnp.int32(0))
        plsc.subcore_barrier()

    pltpu.sync_copy(marker_sh.at[ridx_v.at[pl.ds(0, RB)]], rw_v)
    for v in range(RB // L):
        sl = pl.ds(v * L, L)
        gidx_v[sl] = jnp.bitwise_and(rw_v[sl], jnp.int32(B - 1))
    pltpu.sync_copy(idx_hbm.at[gidx_v], t_v)
    nh = jnp.int32(0)
    for v in range(RB // L):
        sl = pl.ds(v * L, L)
        hit = t_v[sl] == ridx_v[sl]
        hinc = plsc.cumsum(hit.astype(jnp.int32))
        pos = nh + hinc - 1
        rowid = lax.iota(jnp.int32, L) + (v * L)
        plsc.store_scatter(gbuf_v, [pos], gidx_v[sl], mask=hit)
        plsc.store_scatter(dbuf_v, [pos], rowid, mask=hit)
        nh = nh + jnp.sum(hit.astype(jnp.int32), axis=0)

    def drain_mem(j, carry):
        pltpu.make_async_copy(mem_hbm.at[0], out_hbm.at[rbase], sem_rows).wait()
        return carry

    lax.fori_loop(0, RB, drain_mem, jnp.int32(0))

    def patch_row(j, carry):
        g = gbuf_v[pl.ds(j, L)][0]
        d = dbuf_v[pl.ds(j, L)][0]
        pltpu.async_copy(val_hbm.at[g], out_hbm.at[rbase + d], sem_val)
        return carry

    lax.fori_loop(0, nh, patch_row, jnp.int32(0))

    def drain_val(j, carry):
        pltpu.make_async_copy(val_hbm.at[0], out_hbm.at[rbase], sem_val).wait()
        return carry

    lax.fori_loop(0, nh, drain_val, jnp.int32(0))


def kernel(mem, idx, val, read_idx):
    M, D = mem.shape
    B = idx.shape[0]
    assert B % NW == 0 and B & (B - 1) == 0

    RB = B // NW
    WB = B // NS
    mesh = plsc.VectorSubcoreMesh(core_axis_name="c", subcore_axis_name="s")
    body = functools.partial(_sc_body, M, B, D)

    f = pl.kernel(
        body,
        out_type=jax.ShapeDtypeStruct((B, D), jnp.float32),
        mesh=mesh,
        compiler_params=pltpu.CompilerParams(needs_layout_passes=False),
        scratch_types=[
            pltpu.VMEM((WB,), jnp.int32),
            pltpu.VMEM((CC,), jnp.int32),
            pltpu.VMEM((CC,), jnp.int32),
            pltpu.VMEM((2 * WB + RB + L,), jnp.int32),
            pltpu.VMEM((5 * RB + 2 * L,), jnp.int32),
            pltpu.VMEM((2 * (WB + L),), jnp.int32),
            pltpu.VMEM_SHARED((M + CC,), jnp.int32),
            pltpu.SemaphoreType.DMA,
            pltpu.SemaphoreType.DMA,
        ],
    )
    return f(mem, idx, val, read_idx)
